# initial kernel scaffold (unmeasured)
import jax
import jax.numpy as jnp
from jax import lax
from jax.experimental import pallas as pl
from jax.experimental.pallas import tpu as pltpu

N_DEV = 8
B = 2
SEQ = 512
H = 8
D = 64
HIDDEN = 768
NCLS = 4
CLS_ROWS = 128
KV_ROWS = N_DEV * CLS_ROWS


def kernel(x, Wq, K_ext, V_ext, Wo):
    bf16 = jnp.bfloat16

    def body(x_ref, wq_ref, k_ref, v_ref, wo_ref, out_ref,
             comm_ref, kt_ref, vt_ref, qt_ref, ctx_ref,
             send_sems, recv_sems, credit_sem):
        my = lax.axis_index("i")
        left = lax.rem(my + N_DEV - 1, N_DEV)
        right = lax.rem(my + 1, N_DEV)

        barrier = pltpu.get_barrier_semaphore()
        for nbr in (left, right):
            pl.semaphore_signal(barrier, inc=1, device_id=(nbr,),
                                device_id_type=pl.DeviceIdType.MESH)
        pl.semaphore_wait(barrier, 2)

        kt = jnp.swapaxes(k_ref[...].astype(bf16), 1, 2)
        vt = jnp.swapaxes(v_ref[...].astype(bf16), 1, 2)
        for c in range(NCLS):
            comm_ref[0, :, :, c * 128:c * 128 + 64, :] = \
                kt[:, :, c * 64:(c + 1) * 64, :]
            comm_ref[0, :, :, c * 128 + 64:(c + 1) * 128, :] = \
                kt[:, :, (c + 4) * 64:(c + 5) * 64, :]
            comm_ref[0, :, :, 512 + c * 128:512 + c * 128 + 64, :] = \
                vt[:, :, c * 64:(c + 1) * 64, :]
            comm_ref[0, :, :, 512 + c * 128 + 64:512 + (c + 1) * 128, :] = \
                vt[:, :, (c + 4) * 64:(c + 5) * 64, :]

        def scatter(slot, s):
            off = s * CLS_ROWS
            for c in range(NCLS):
                kt_ref[c, :, :, pl.ds(off, CLS_ROWS), :] = \
                    comm_ref[slot, :, :, c * 128:(c + 1) * 128, :]
                vt_ref[c, :, :, pl.ds(off, CLS_ROWS), :] = \
                    comm_ref[slot, :, :, 512 + c * 128:512 + (c + 1) * 128, :]

        scatter(0, my)

        xq = jnp.reshape(x_ref[...], (B * SEQ, HIDDEN)).astype(bf16)
        q = lax.dot_general(xq, wq_ref[...].astype(bf16),
                            (((1,), (0,)), ((), ())),
                            preferred_element_type=jnp.float32)
        q = (q * 0.125).astype(bf16)
        qt_ref[...] = jnp.swapaxes(jnp.reshape(q, (B, SEQ, H, D)), 1, 2)

        for hop in range(N_DEV - 1):
            send_slot = hop % 2
            recv_slot = (hop + 1) % 2
            if hop >= 2:
                pl.semaphore_wait(credit_sem, 1)
            rdma = pltpu.make_async_remote_copy(
                src_ref=comm_ref.at[send_slot],
                dst_ref=comm_ref.at[recv_slot],
                send_sem=send_sems.at[send_slot],
                recv_sem=recv_sems.at[recv_slot],
                device_id=(right,),
                device_id_type=pl.DeviceIdType.MESH,
            )
            rdma.start()
            rdma.wait()
            s = lax.rem(my + (N_DEV - 1 - hop), N_DEV)
            scatter(recv_slot, s)
            if hop <= N_DEV - 4:
                pl.semaphore_signal(credit_sem, inc=1, device_id=(left,),
                                    device_id_type=pl.DeviceIdType.MESH)

        def attn_body(i, carry):
            b = i // H
            hh = lax.rem(i, H)
            for c in range(NCLS):
                q0 = qt_ref[b, hh, c * 64:(c + 1) * 64, :]
                q1 = qt_ref[b, hh, (c + 4) * 64:(c + 5) * 64, :]
                qc = jnp.concatenate([q0, q1], axis=0)
                kc = kt_ref[c, b, hh]
                sc = lax.dot_general(qc, kc, (((1,), (1,)), ((), ())),
                                     preferred_element_type=jnp.float32)
                mx = jnp.max(sc, axis=1, keepdims=True)
                p = jnp.exp(sc - mx)
                w = (p / jnp.sum(p, axis=1, keepdims=True)).astype(bf16)
                ctxb = lax.dot_general(w, vt_ref[c, b, hh],
                                       (((1,), (0,)), ((), ())),
                                       preferred_element_type=jnp.float32)
                ctxb = ctxb.astype(bf16)
                ctx_ref[b, hh, c * 64:(c + 1) * 64, :] = ctxb[:64]
                ctx_ref[b, hh, (c + 4) * 64:(c + 5) * 64, :] = ctxb[64:]
            return carry

        lax.fori_loop(0, B * H, attn_body, 0)

        ctxv = jnp.reshape(jnp.swapaxes(ctx_ref[...], 1, 2), (B * SEQ, H * D))
        o = lax.dot_general(ctxv, wo_ref[...].astype(bf16),
                            (((1,), (0,)), ((), ())),
                            preferred_element_type=jnp.float32)
        out_ref[...] = jnp.reshape(o, (B, SEQ, HIDDEN))

    return pl.pallas_call(
        body,
        out_shape=jax.ShapeDtypeStruct((B, SEQ, HIDDEN), jnp.float32),
        in_specs=[pl.BlockSpec(memory_space=pltpu.VMEM)] * 5,
        out_specs=pl.BlockSpec(memory_space=pltpu.VMEM),
        scratch_shapes=[
            pltpu.VMEM((2, B, H, 2 * SEQ, D), bf16),
            pltpu.VMEM((NCLS, B, H, KV_ROWS, D), bf16),
            pltpu.VMEM((NCLS, B, H, KV_ROWS, D), bf16),
            pltpu.VMEM((B, H, SEQ, D), bf16),
            pltpu.VMEM((B, H, SEQ, D), bf16),
            pltpu.SemaphoreType.DMA((2,)),
            pltpu.SemaphoreType.DMA((2,)),
            pltpu.SemaphoreType.REGULAR,
        ],
        compiler_params=pltpu.CompilerParams(collective_id=0),
    )(x, Wq, K_ext, V_ext, Wo)


# baseline (device time: 391249 ns/iter reference)
import jax
import jax.numpy as jnp
from jax import lax
from jax.experimental import pallas as pl
from jax.experimental.pallas import tpu as pltpu

N_DEV = 8
B = 2
SEQ = 512
H = 8
D = 64
HIDDEN = 768
NCLS = 4
CLS_ROWS = 128


def kernel(x, Wq, K_ext, V_ext, Wo):
    bf16 = jnp.bfloat16

    def body(x_hbm, wq_ref, k_hbm, v_hbm, wo_ref, out_ref,
             comm_ref, qt_ref, ctx_ref, stage_kv, stage_x,
             send_sems, recv_sems, copy_sem):
        my = lax.axis_index("i")
        left = lax.rem(my + N_DEV - 1, N_DEV)
        right = lax.rem(my + 1, N_DEV)

        barrier = pltpu.get_barrier_semaphore()
        for nbr in (left, right):
            pl.semaphore_signal(barrier, inc=1, device_id=(nbr,),
                                device_id_type=pl.DeviceIdType.MESH)
        pl.semaphore_wait(barrier, 2)

        def load_kv_into_slot(src_hbm, row0):
            cp = pltpu.make_async_copy(src_hbm, stage_kv, copy_sem)
            cp.start()
            cp.wait()
            t = jnp.swapaxes(stage_kv[...].astype(bf16), 1, 2)
            for c in range(NCLS):
                comm_ref[my, :, :, pl.ds(row0 + c * 128, 64), :] = \
                    t[:, :, c * 64:(c + 1) * 64, :]
                comm_ref[my, :, :, pl.ds(row0 + c * 128 + 64, 64), :] = \
                    t[:, :, (c + 4) * 64:(c + 5) * 64, :]

        load_kv_into_slot(k_hbm, 0)
        load_kv_into_slot(v_hbm, SEQ)

        wq_bf = wq_ref[...].astype(bf16)
        for b in range(B):
            cp = pltpu.make_async_copy(x_hbm.at[b], stage_x, copy_sem)
            cp.start()
            cp.wait()
            q = lax.dot_general(stage_x[...].astype(bf16), wq_bf,
                                (((1,), (0,)), ((), ())),
                                preferred_element_type=jnp.float32)
            q = (q * 0.125).astype(bf16)
            qt_ref[b] = jnp.swapaxes(jnp.reshape(q, (SEQ, H, D)), 0, 1)

        for hop in range(N_DEV - 1):
            o = lax.rem(my - hop + N_DEV, N_DEV)
            rdma = pltpu.make_async_remote_copy(
                src_ref=comm_ref.at[o],
                dst_ref=comm_ref.at[o],
                send_sem=send_sems.at[hop],
                recv_sem=recv_sems.at[hop],
                device_id=(right,),
                device_id_type=pl.DeviceIdType.MESH,
            )
            rdma.start()
            rdma.wait()

        def attn_body(i, carry):
            b = i // H
            hh = lax.rem(i, H)
            for c in range(NCLS):
                qc = jnp.concatenate(
                    [qt_ref[b, hh, c * 64:(c + 1) * 64, :],
                     qt_ref[b, hh, (c + 4) * 64:(c + 5) * 64, :]], axis=0)
                kc = jnp.concatenate(
                    [comm_ref[s, b, hh, c * 128:(c + 1) * 128, :]
                     for s in range(N_DEV)], axis=0)
                vc = jnp.concatenate(
                    [comm_ref[s, b, hh,
                              SEQ + c * 128:SEQ + (c + 1) * 128, :]
                     for s in range(N_DEV)], axis=0)
                sc = lax.dot_general(qc, kc, (((1,), (1,)), ((), ())),
                                     preferred_element_type=jnp.float32)
                mx = jnp.max(sc, axis=1, keepdims=True)
                p = jnp.exp(sc - mx)
                w = (p / jnp.sum(p, axis=1, keepdims=True)).astype(bf16)
                ctxb = lax.dot_general(w, vc, (((1,), (0,)), ((), ())),
                                       preferred_element_type=jnp.float32)
                ctxb = ctxb.astype(bf16)
                ctx_ref[b, hh, c * 64:(c + 1) * 64, :] = ctxb[:64]
                ctx_ref[b, hh, (c + 4) * 64:(c + 5) * 64, :] = ctxb[64:]
            return carry

        lax.fori_loop(0, B * H, attn_body, 0)

        wo_bf = wo_ref[...].astype(bf16)
        for b in range(B):
            ctxb = jnp.reshape(jnp.swapaxes(ctx_ref[b], 0, 1), (SEQ, H * D))
            out_ref[b] = lax.dot_general(ctxb, wo_bf,
                                         (((1,), (0,)), ((), ())),
                                         preferred_element_type=jnp.float32)

    return pl.pallas_call(
        body,
        out_shape=jax.ShapeDtypeStruct((B, SEQ, HIDDEN), jnp.float32),
        in_specs=[
            pl.BlockSpec(memory_space=pltpu.MemorySpace.HBM),
            pl.BlockSpec(memory_space=pltpu.VMEM),
            pl.BlockSpec(memory_space=pltpu.MemorySpace.HBM),
            pl.BlockSpec(memory_space=pltpu.MemorySpace.HBM),
            pl.BlockSpec(memory_space=pltpu.VMEM),
        ],
        out_specs=pl.BlockSpec(memory_space=pltpu.VMEM),
        scratch_shapes=[
            pltpu.VMEM((N_DEV, B, H, 2 * SEQ, D), bf16),
            pltpu.VMEM((B, H, SEQ, D), bf16),
            pltpu.VMEM((B, H, SEQ, D), bf16),
            pltpu.VMEM((B, SEQ, H, D), jnp.float32),
            pltpu.VMEM((SEQ, HIDDEN), jnp.float32),
            pltpu.SemaphoreType.DMA((N_DEV - 1,)),
            pltpu.SemaphoreType.DMA((N_DEV - 1,)),
            pltpu.SemaphoreType.DMA,
        ],
        compiler_params=pltpu.CompilerParams(
            collective_id=0,
            vmem_limit_bytes=100 * 1024 * 1024,
        ),
    )(x, Wq, K_ext, V_ext, Wo)
